# baseline (speedup 1.0000x reference)
import functools

import jax
import jax.numpy as jnp
from jax.experimental import pallas as pl
from jax.experimental.pallas import tpu as pltpu

_DIM = 64
_N_EMBED = 1024
_ROWS = 16
_COLS = 1024
_TOKENS = _ROWS * _COLS
_BR = 1
_BLK = _BR * _COLS
_NUM_CHUNKS = _ROWS // _BR


def _vq_body(x_hbm, e_ref, q_hbm, ind_ref, diff_ref, ppl_ref,
             x_buf, q_buf, sem_x, sem_q):
    e = e_ref[...]
    e_sq = jnp.sum(e * e, axis=0, keepdims=True)
    iota = jax.lax.broadcasted_iota(jnp.int32, (_BLK, _N_EMBED), 1)

    def x_copy(c):
        return pltpu.make_async_copy(
            x_hbm.at[pl.ds(c * _BR, _BR)], x_buf.at[pl.ds(c % 2, 1)],
            sem_x.at[c % 2])

    def q_copy(c):
        return pltpu.make_async_copy(
            q_buf.at[pl.ds(c % 2, 1)], q_hbm.at[pl.ds(c * _BR, _BR)],
            sem_q.at[c % 2])

    x_copy(0).start()
    cnt = jnp.zeros((_N_EMBED,), dtype=jnp.float32)
    dsum = jnp.float32(0.0)
    for c in range(_NUM_CHUNKS):
        if c + 1 < _NUM_CHUNKS:
            x_copy(c + 1).start()
        x_copy(c).wait()
        x = x_buf[c % 2].reshape(_BLK, _DIM)
        neg2_scores = jax.lax.dot_general(
            x * (-2.0), e, (((1,), (0,)), ((), ())),
            preferred_element_type=jnp.float32)
        x_sq = jnp.sum(x * x, axis=1, keepdims=True)
        dist = (x_sq + neg2_scores) + e_sq

        ind = jnp.argmin(dist, axis=1).astype(jnp.int32)
        onehot = (iota == ind[:, None]).astype(jnp.float32)
        q = jax.lax.dot_general(
            onehot, e, (((1,), (1,)), ((), ())),
            preferred_element_type=jnp.float32)

        if c >= 2:
            q_copy(c - 2).wait()
        q_buf[c % 2] = x + (q - x)
        q_copy(c).start()
        ind_ref[c * _BLK:(c + 1) * _BLK] = ind

        ones = jnp.ones((1, _BLK), dtype=jnp.float32)
        cnt = cnt + jax.lax.dot_general(
            ones, onehot, (((1,), (0,)), ((), ())),
            preferred_element_type=jnp.float32)[0]
        dsum = dsum + jnp.sum((q - x) ** 2)

    diff_ref[...] = jnp.reshape(dsum / float(_TOKENS * _DIM), (1, 1))
    p = cnt / float(_TOKENS)
    ent = jnp.sum(p * jnp.log(jnp.clip(p, 1e-7, None)), keepdims=True)
    ppl_ref[...] = jnp.exp(-ent).reshape(1, 1)
    q_copy(_NUM_CHUNKS - 2).wait()
    q_copy(_NUM_CHUNKS - 1).wait()


@functools.partial(jax.jit, static_argnames=())
def kernel(input, embed):
    flat = input.reshape(_ROWS // _BR, _BR * _COLS, _DIM)
    q, ind, diff, ppl = pl.pallas_call(
        _vq_body,
        in_specs=[
            pl.BlockSpec(memory_space=pltpu.MemorySpace.HBM),
            pl.BlockSpec(memory_space=pltpu.MemorySpace.VMEM),
        ],
        out_specs=[
            pl.BlockSpec(memory_space=pltpu.MemorySpace.HBM),
            pl.BlockSpec(memory_space=pltpu.MemorySpace.VMEM),
            pl.BlockSpec(memory_space=pltpu.MemorySpace.VMEM),
            pl.BlockSpec(memory_space=pltpu.MemorySpace.VMEM),
        ],
        out_shape=[
            jax.ShapeDtypeStruct((_ROWS // _BR, _BR * _COLS, _DIM), jnp.float32),
            jax.ShapeDtypeStruct((_TOKENS,), jnp.int32),
            jax.ShapeDtypeStruct((1, 1), jnp.float32),
            jax.ShapeDtypeStruct((1, 1), jnp.float32),
        ],
        scratch_shapes=[
            pltpu.VMEM((2, _BLK, _DIM), jnp.float32),
            pltpu.VMEM((2, _BLK, _DIM), jnp.float32),
            pltpu.SemaphoreType.DMA((2,)),
            pltpu.SemaphoreType.DMA((2,)),
        ],
    )(flat, embed)
    return (q.reshape(_ROWS, _COLS, _DIM), diff[0, 0],
            ind.reshape(_ROWS, _COLS), ppl[0, 0])


# speedup vs baseline: 1.4234x; 1.4234x over previous
import functools

import jax
import jax.numpy as jnp
from jax.experimental import pallas as pl
from jax.experimental.pallas import tpu as pltpu

_DIM = 64
_N_EMBED = 1024
_ROWS = 16
_COLS = 1024
_TOKENS = _ROWS * _COLS
_BR = 1
_BLK = _BR * _COLS
_NUM_CHUNKS = _ROWS // _BR


def _vq_body(x_ref, e_ref, q_ref, ind_ref, diff_ref, ppl_ref):
    e = e_ref[...]
    e_sq = jnp.sum(e * e, axis=0, keepdims=True)
    iota = jax.lax.broadcasted_iota(jnp.int32, (_BLK, _N_EMBED), 1)

    cnt = jnp.zeros((_N_EMBED,), dtype=jnp.float32)
    dsum = jnp.float32(0.0)
    for c in range(_NUM_CHUNKS):
        x = x_ref[c * _BR:(c + 1) * _BR].reshape(_BLK, _DIM)
        neg2_scores = jax.lax.dot_general(
            x * (-2.0), e, (((1,), (0,)), ((), ())),
            preferred_element_type=jnp.float32)
        x_sq = jnp.sum(x * x, axis=1, keepdims=True)
        dist = (x_sq + neg2_scores) + e_sq

        ind = jnp.argmin(dist, axis=1).astype(jnp.int32)
        onehot = (iota == ind[:, None]).astype(jnp.float32)
        q = jax.lax.dot_general(
            onehot, e, (((1,), (1,)), ((), ())),
            preferred_element_type=jnp.float32)

        q_ref[c * _BR:(c + 1) * _BR] = q.reshape(_BR, _COLS, _DIM)
        ind_ref[c * _BLK:(c + 1) * _BLK] = ind

        ones = jnp.ones((1, _BLK), dtype=jnp.float32)
        cnt = cnt + jax.lax.dot_general(
            ones, onehot, (((1,), (0,)), ((), ())),
            preferred_element_type=jnp.float32)[0]
        dsum = dsum + jnp.sum((q - x) ** 2)

    diff_ref[...] = jnp.reshape(dsum / float(_TOKENS * _DIM), (1, 1))
    p = cnt / float(_TOKENS)
    ent = jnp.sum(p * jnp.log(jnp.clip(p, 1e-7, None)), keepdims=True)
    ppl_ref[...] = jnp.exp(-ent).reshape(1, 1)


@functools.partial(jax.jit, static_argnames=())
def kernel(input, embed):
    q, ind, diff, ppl = pl.pallas_call(
        _vq_body,
        out_shape=[
            jax.ShapeDtypeStruct((_ROWS, _COLS, _DIM), jnp.float32),
            jax.ShapeDtypeStruct((_TOKENS,), jnp.int32),
            jax.ShapeDtypeStruct((1, 1), jnp.float32),
            jax.ShapeDtypeStruct((1, 1), jnp.float32),
        ],
    )(input, embed)
    return q, diff[0, 0], ind.reshape(_ROWS, _COLS), ppl[0, 0]


# speedup vs baseline: 1.4475x; 1.0170x over previous
import functools

import jax
import jax.numpy as jnp
from jax.experimental import pallas as pl
from jax.experimental.pallas import tpu as pltpu

_DIM = 64
_N_EMBED = 1024
_ROWS = 16
_COLS = 1024
_TOKENS = _ROWS * _COLS
_BR = 1
_BLK = _BR * _COLS
_NUM_CHUNKS = _ROWS // _BR


def _vq_body(x_ref, e_ref, q_ref, ind_ref, diff_ref, ppl_ref):
    e = e_ref[...]
    e_sq = jnp.sum(e * e, axis=0, keepdims=True)
    iota = jax.lax.broadcasted_iota(jnp.int32, (_BLK, _N_EMBED), 1)

    cnt = jnp.zeros((_N_EMBED,), dtype=jnp.float32)
    dsum = jnp.float32(0.0)
    for c in range(_NUM_CHUNKS):
        x = x_ref[c * _BR:(c + 1) * _BR].reshape(_BLK, _DIM)
        neg2_scores = jax.lax.dot_general(
            x * (-2.0), e, (((1,), (0,)), ((), ())),
            preferred_element_type=jnp.float32)
        x_sq = jnp.sum(x * x, axis=1, keepdims=True)
        dist = (x_sq + neg2_scores) + e_sq

        ind = jnp.argmin(dist, axis=1).astype(jnp.int32)
        onehot = (iota == ind[:, None]).astype(jnp.float32)
        q = jax.lax.dot_general(
            onehot, e, (((1,), (1,)), ((), ())),
            preferred_element_type=jnp.float32)

        q_ref[c * _BR:(c + 1) * _BR] = q.reshape(_BR, _COLS, _DIM)
        ind_ref[c * _BLK:(c + 1) * _BLK] = ind

        ones = jnp.ones((1, _BLK), dtype=jnp.float32)
        cnt = cnt + jax.lax.dot_general(
            ones, onehot, (((1,), (0,)), ((), ())),
            preferred_element_type=jnp.float32)[0]
        dsum = dsum + jnp.sum((q - x) ** 2)

    diff_ref[0] = dsum / float(_TOKENS * _DIM)
    p = cnt / float(_TOKENS)
    ent = jnp.sum(p * jnp.log(jnp.clip(p, 1e-7, None)))
    ppl_ref[0] = jnp.exp(-ent)


@functools.partial(jax.jit, static_argnames=())
def kernel(input, embed):
    q, ind, diff, ppl = pl.pallas_call(
        _vq_body,
        out_specs=[
            pl.BlockSpec(memory_space=pltpu.MemorySpace.VMEM),
            pl.BlockSpec(memory_space=pltpu.MemorySpace.VMEM),
            pl.BlockSpec(memory_space=pltpu.MemorySpace.SMEM),
            pl.BlockSpec(memory_space=pltpu.MemorySpace.SMEM),
        ],
        out_shape=[
            jax.ShapeDtypeStruct((_ROWS, _COLS, _DIM), jnp.float32),
            jax.ShapeDtypeStruct((_TOKENS,), jnp.int32),
            jax.ShapeDtypeStruct((1,), jnp.float32),
            jax.ShapeDtypeStruct((1,), jnp.float32),
        ],
    )(input, embed)
    return q, diff[0], ind.reshape(_ROWS, _COLS), ppl[0]
